# Initial kernel scaffold; baseline (speedup 1.0000x reference)
import jax
import jax.numpy as jnp
from jax.experimental import pallas as pl


def kernel(x, H, W, wg_w, wg_b, fc1_w, fc1_b, dw_w, dw_b, fc2_w, fc2_b):
    raise NotImplementedError("write your pallas kernel here")


# speedup vs baseline: 13.2423x; 13.2423x over previous
import jax
import jax.numpy as jnp
from jax.experimental import pallas as pl
from jax.experimental.pallas import tpu as pltpu

_E = 8
_DIM = 96
_HID = 384
_OUT = 96
_B = 2
_N = 1024
_NT = _B * _N


def _col_to_row(col):
    i = jax.lax.broadcasted_iota(jnp.int32, (_E, _E), 0)
    j = jax.lax.broadcasted_iota(jnp.int32, (_E, _E), 1)
    b = jnp.broadcast_to(col, (_E, _E))
    return jnp.sum(jnp.where(i == j, b, 0.0), axis=0, keepdims=True)


def _row_to_col(row):
    i = jax.lax.broadcasted_iota(jnp.int32, (_E, _E), 0)
    j = jax.lax.broadcasted_iota(jnp.int32, (_E, _E), 1)
    b = jnp.broadcast_to(row, (_E, _E))
    return jnp.sum(jnp.where(i == j, b, 0.0), axis=1, keepdims=True)


def _shift_down(a):
    return jnp.concatenate([jnp.zeros((1, _HID), jnp.float32), a[:-1]], axis=0)


def _shift_up(a):
    return jnp.concatenate([a[1:], jnp.zeros((1, _HID), jnp.float32)], axis=0)


def _moe_kernel(x_ref, wgt_ref, wgb_ref, fc1t_ref, fc1b_ref, dwt_ref,
                dwb_ref, fc2t_ref, fc2b_ref, out_ref):
    x = x_ref[...]
    logits = jnp.dot(x, wgt_ref[...], preferred_element_type=jnp.float32)
    logits = logits + wgb_ref[...]
    mx = jnp.max(logits, axis=1, keepdims=True)
    iota_e = jax.lax.broadcasted_iota(jnp.int32, (_NT, _E), 1)
    idx = jnp.min(jnp.where(logits == mx, iota_e, _E), axis=1, keepdims=True)
    onehot = jnp.where(iota_e == idx, 1.0, 0.0)
    pres0 = jnp.max(onehot[:_N], axis=0, keepdims=True)
    pres1 = jnp.max(onehot[_N:], axis=0, keepdims=True)
    ex = jnp.exp(logits - mx)
    p0 = ex[:, 0:1] / jnp.sum(ex, axis=1, keepdims=True)
    p0_b0 = _col_to_row(p0[0:_E])
    p0_b1 = _col_to_row(p0[_N:_N + _E])
    masked0 = p0_b0 * pres0
    masked1 = p0_b1 * pres1
    denom = masked0 + masked1 + 1e-6
    gs0 = masked0 / denom * float(_B)
    gs1 = masked1 / denom * float(_B)
    gs_col = jnp.concatenate([_row_to_col(gs0), _row_to_col(gs1)], axis=0)
    x64 = jnp.concatenate([x[0:64], x[_N:_N + 64]], axis=0)
    h = jnp.dot(x64, fc1t_ref[...], preferred_element_type=jnp.float32)
    h = h + fc1b_ref[...]
    taps = dwt_ref[...]
    outs = []
    for b in range(_B):
        r0 = h[b * 64:b * 64 + 32]
        r1 = h[b * 64 + 32:b * 64 + 64]
        conv = (_shift_down(r0) * taps[0:1] + r0 * taps[1:2]
                + _shift_up(r0) * taps[2:3]
                + _shift_down(r1) * taps[3:4] + r1 * taps[4:5]
                + _shift_up(r1) * taps[5:6]) + dwb_ref[...]
        outs.append(conv[0:_E])
    g = jnp.concatenate(outs, axis=0)
    g = 0.5 * g * (1.0 + jax.lax.erf(g * 0.7071067811865476))
    y = jnp.dot(g, fc2t_ref[...], preferred_element_type=jnp.float32)
    y = (y + fc2b_ref[...]) * gs_col
    out_ref[...] = jnp.zeros((_NT, _OUT), jnp.float32)
    out_ref[0:_E, :] = y[0:_E]
    out_ref[_N:_N + _E, :] = y[_E:2 * _E]


def kernel(x, H, W, wg_w, wg_b, fc1_w, fc1_b, dw_w, dw_b, fc2_w, fc2_b):
    xf = x.reshape(_NT, _DIM)
    wgt = wg_w.T
    wgb = wg_b.reshape(1, _E)
    fc1t = fc1_w[0].T
    fc1b = fc1_b[0].reshape(1, _HID)
    dwt = dw_w[0, :, 0].reshape(_HID, 9).T[3:9]
    dwb = dw_b[0].reshape(1, _HID)
    fc2t = fc2_w[0].T
    fc2b = fc2_b[0].reshape(1, _OUT)
    out = pl.pallas_call(
        _moe_kernel,
        out_shape=jax.ShapeDtypeStruct((_NT, _OUT), jnp.float32),
    )(xf, wgt, wgb, fc1t, fc1b, dwt, dwb, fc2t, fc2b)
    return (out.reshape(_B, _N, _OUT), None)


# speedup vs baseline: 14.8329x; 1.1201x over previous
import jax
import jax.numpy as jnp
from jax.experimental import pallas as pl
from jax.experimental.pallas import tpu as pltpu

_E = 8
_DIM = 96
_HID = 384
_OUT = 96
_B = 2
_N = 1024
_NT = _B * _N

_CT = (((1,), (1,)), ((), ()))


def _col_to_row(col):
    i = jax.lax.broadcasted_iota(jnp.int32, (_E, _E), 0)
    j = jax.lax.broadcasted_iota(jnp.int32, (_E, _E), 1)
    b = jnp.broadcast_to(col, (_E, _E))
    return jnp.sum(jnp.where(i == j, b, 0.0), axis=0, keepdims=True)


def _row_to_col(row):
    i = jax.lax.broadcasted_iota(jnp.int32, (_E, _E), 0)
    j = jax.lax.broadcasted_iota(jnp.int32, (_E, _E), 1)
    b = jnp.broadcast_to(row, (_E, _E))
    return jnp.sum(jnp.where(i == j, b, 0.0), axis=1, keepdims=True)


def _shift_down(a):
    return jnp.concatenate([jnp.zeros((1, _HID), jnp.float32), a[:-1]], axis=0)


def _shift_up(a):
    return jnp.concatenate([a[1:], jnp.zeros((1, _HID), jnp.float32)], axis=0)


def _moe_kernel(x_ref, wg_ref, wgb_ref, fc1_ref, fc1b_ref, dw_ref,
                dwb_ref, fc2_ref, fc2b_ref, out_ref):
    x = x_ref[...]
    logits = jax.lax.dot_general(x, wg_ref[...], _CT,
                                 preferred_element_type=jnp.float32)
    logits = logits + wgb_ref[...]
    mx = jnp.max(logits, axis=1, keepdims=True)
    iota_e = jax.lax.broadcasted_iota(jnp.int32, (_NT, _E), 1)
    idx = jnp.min(jnp.where(logits == mx, iota_e, _E), axis=1, keepdims=True)
    onehot = jnp.where(iota_e == idx, 1.0, 0.0)
    pres0 = jnp.max(onehot[:_N], axis=0, keepdims=True)
    pres1 = jnp.max(onehot[_N:], axis=0, keepdims=True)
    ex = jnp.exp(logits - mx)
    p0 = ex[:, 0:1] / jnp.sum(ex, axis=1, keepdims=True)
    p0_b0 = _col_to_row(p0[0:_E])
    p0_b1 = _col_to_row(p0[_N:_N + _E])
    masked0 = p0_b0 * pres0
    masked1 = p0_b1 * pres1
    denom = masked0 + masked1 + 1e-6
    gs0 = masked0 / denom * float(_B)
    gs1 = masked1 / denom * float(_B)
    gs_col = jnp.concatenate([_row_to_col(gs0), _row_to_col(gs1)], axis=0)
    x64 = jnp.concatenate([x[0:64], x[_N:_N + 64]], axis=0)
    h = jax.lax.dot_general(x64, fc1_ref[0], _CT,
                            preferred_element_type=jnp.float32)
    h = h + fc1b_ref[0]
    taps = jnp.transpose(dw_ref[0])
    outs = []
    for b in range(_B):
        r0 = h[b * 64:b * 64 + 32]
        r1 = h[b * 64 + 32:b * 64 + 64]
        conv = (_shift_down(r0) * taps[3:4] + r0 * taps[4:5]
                + _shift_up(r0) * taps[5:6]
                + _shift_down(r1) * taps[6:7] + r1 * taps[7:8]
                + _shift_up(r1) * taps[8:9]) + dwb_ref[0]
        outs.append(conv[0:_E])
    g = jnp.concatenate(outs, axis=0)
    g = 0.5 * g * (1.0 + jax.lax.erf(g * 0.7071067811865476))
    y = jax.lax.dot_general(g, fc2_ref[0], _CT,
                            preferred_element_type=jnp.float32)
    y = (y + fc2b_ref[0]) * gs_col
    out_ref[...] = jnp.zeros((_NT, _OUT), jnp.float32)
    out_ref[0:_E, :] = y[0:_E]
    out_ref[_N:_N + _E, :] = y[_E:2 * _E]


def kernel(x, H, W, wg_w, wg_b, fc1_w, fc1_b, dw_w, dw_b, fc2_w, fc2_b):
    xf = x.reshape(_NT, _DIM)
    wgb = wg_b.reshape(1, _E)
    fc1b = fc1_b.reshape(_E, 1, _HID)
    dwf = dw_w.reshape(_E, _HID, 9)
    dwb = dw_b.reshape(_E, 1, _HID)
    fc2b = fc2_b.reshape(_E, 1, _OUT)
    out = pl.pallas_call(
        _moe_kernel,
        grid=(1,),
        in_specs=[
            pl.BlockSpec((_NT, _DIM), lambda i: (0, 0)),
            pl.BlockSpec((_E, _DIM), lambda i: (0, 0)),
            pl.BlockSpec((1, _E), lambda i: (0, 0)),
            pl.BlockSpec((1, _HID, _DIM), lambda i: (0, 0, 0)),
            pl.BlockSpec((1, 1, _HID), lambda i: (0, 0, 0)),
            pl.BlockSpec((1, _HID, 9), lambda i: (0, 0, 0)),
            pl.BlockSpec((1, 1, _HID), lambda i: (0, 0, 0)),
            pl.BlockSpec((1, _OUT, _HID), lambda i: (0, 0, 0)),
            pl.BlockSpec((1, 1, _OUT), lambda i: (0, 0, 0)),
        ],
        out_specs=pl.BlockSpec((_NT, _OUT), lambda i: (0, 0)),
        out_shape=jax.ShapeDtypeStruct((_NT, _OUT), jnp.float32),
    )(xf, wg_w, wgb, fc1_w, fc1b, dwf, dwb, fc2_w, fc2b)
    return (out.reshape(_B, _N, _OUT), None)
